# baseline (speedup 1.0000x reference)
import functools

import jax
import jax.numpy as jnp
from jax.experimental import pallas as pl
from jax.experimental.pallas import tpu as pltpu


def _dist_kernel(f_ref, c_ref, o_ref, cs_ref, c2_ref):
    @pl.when(pl.program_id(0) == 0)
    def _():
        c = c_ref[...]
        cs_ref[...] = (-2.0 * c).astype(jnp.bfloat16)
        c2_ref[0:1, :] = jnp.sum(c * c, axis=1)[None, :]

    f = f_ref[...]
    f2 = jnp.sum(f * f, axis=1, keepdims=True)
    fc = jax.lax.dot_general(
        f.astype(jnp.bfloat16), cs_ref[...],
        (((1,), (1,)), ((), ())),
        preferred_element_type=jnp.float32,
    )
    o_ref[...] = (fc + f2) + c2_ref[0:1, :]


@functools.partial(jax.jit, static_argnames=("bm",))
def _dists(f, c, bm):
    M, D = f.shape
    K = c.shape[0]
    grid = (M // bm,)
    return pl.pallas_call(
        _dist_kernel,
        grid=grid,
        in_specs=[
            pl.BlockSpec((bm, D), lambda i: (i, 0)),
            pl.BlockSpec((K, D), lambda i: (0, 0)),
        ],
        out_specs=pl.BlockSpec((bm, K), lambda i: (i, 0)),
        out_shape=jax.ShapeDtypeStruct((M, K), jnp.float32),
        scratch_shapes=[
            pltpu.VMEM((K, D), jnp.bfloat16),
            pltpu.VMEM((8, K), jnp.float32),
        ],
        compiler_params=pltpu.CompilerParams(
            dimension_semantics=("arbitrary",),
        ),
    )(f, c)


def kernel(features, Ck):
    B, S, D = features.shape
    K = Ck.shape[1]
    f = features.reshape(B * S, D)
    c = Ck.reshape(K, D)
    dists = _dists(f, c, bm=2304)
    return dists.reshape(B, S, K)
